# Initial kernel scaffold; baseline (speedup 1.0000x reference)
import jax
import jax.numpy as jnp
from jax.experimental import pallas as pl


def kernel(x, c, edge_index, Wx1, Wm1, Wc1, b1, Wcc1, bc1, Wx2, Wm2, Wc2, b2, Wcc2, bc2):
    raise NotImplementedError("write your pallas kernel here")


# speedup vs baseline: 37.7358x; 37.7358x over previous
import functools

import jax
import jax.numpy as jnp
from jax import lax
from jax.experimental import pallas as pl
from jax.experimental.pallas import tpu as pltpu
from jax.experimental.pallas import tpu_sc as plsc

_NCORES = 2
_NSUB = 16
_NW = _NCORES * _NSUB
_BLK = 128
_B2 = 8


def _make_edge_scatter(npad: int, feat: int, n_pairs: int):
  mesh = plsc.VectorSubcoreMesh(core_axis_name="c", subcore_axis_name="s")
  stripe = npad // _NSUB

  @functools.partial(
      pl.kernel,
      out_type=jax.ShapeDtypeStruct((_NCORES, npad, feat), jnp.float32),
      mesh=mesh,
      compiler_params=pltpu.CompilerParams(use_tc_tiling_on_sc=False),
      scratch_types=[
          pltpu.VMEM((2, _B2, _BLK), jnp.int32),
          pltpu.VMEM((2, _B2, _BLK), jnp.int32),
          pltpu.VMEM((2, _B2, _BLK, feat), jnp.float32),
          pltpu.VMEM_SHARED((npad, feat), jnp.float32),
          pltpu.SemaphoreType.DMA,
          pltpu.SemaphoreType.DMA,
      ],
  )
  def scatter_kernel(x_hbm, src_hbm, dst_hbm, z_hbm, out_hbm,
                     src_v, dst_v, rows_v, acc, gsem, ssem):
    cid = lax.axis_index("c")
    sid = lax.axis_index("s")
    wid = cid * _NSUB + sid

    pltpu.sync_copy(z_hbm.at[pl.ds(sid * stripe, stripe)],
                    acc.at[pl.ds(sid * stripe, stripe)])
    plsc.subcore_barrier()

    base = wid * (n_pairs * 2 * _B2)

    def load_idx(slot, row):
      pltpu.sync_copy(src_hbm.at[pl.ds(row, _B2)], src_v.at[slot])
      pltpu.sync_copy(dst_hbm.at[pl.ds(row, _B2)], dst_v.at[slot])

    def fire_gathers(slot):
      for j in range(_B2):
        pltpu.async_copy(x_hbm.at[src_v.at[slot, j]], rows_v.at[slot, j],
                         gsem)

    def wait_gathers(slot):
      for j in range(_B2):
        pltpu.make_async_copy(x_hbm.at[src_v.at[slot, j]],
                              rows_v.at[slot, j], gsem).wait()

    def fire_scatters(slot):
      for j in range(_B2):
        pltpu.async_copy(rows_v.at[slot, j], acc.at[dst_v.at[slot, j]],
                         ssem, add=True)

    def drain_scatters(slot):
      for j in range(_B2):
        pltpu.make_async_copy(rows_v.at[slot, j],
                              acc.at[dst_v.at[slot, j]], ssem).wait()

    load_idx(0, base)
    fire_gathers(0)

    def body(p, carry):
      row = base + p * 2 * _B2
      wait_gathers(0)
      fire_scatters(0)

      @pl.when(p > 0)
      def _():
        drain_scatters(1)
      load_idx(1, row + _B2)
      fire_gathers(1)

      wait_gathers(1)
      fire_scatters(1)
      drain_scatters(0)

      @pl.when(p < n_pairs - 1)
      def _():
        load_idx(0, row + 2 * _B2)
        fire_gathers(0)
      return carry

    lax.fori_loop(0, n_pairs, body, 0)
    drain_scatters(1)

    plsc.subcore_barrier()
    pltpu.sync_copy(acc.at[pl.ds(sid * stripe, stripe)],
                    out_hbm.at[cid, pl.ds(sid * stripe, stripe)])

  return scatter_kernel


def _tc_layer1(n: int, bn: int):
  def body(x_r, c_r, p0_r, p1_r, wx_r, wm_r, wc_r, b_r, wcc_r, bc_r, wm2_r,
           h_r, ch_r, g_r):
    agg = p0_r[0] + p1_r[0]
    h = (jnp.dot(x_r[...], wx_r[...], preferred_element_type=jnp.float32)
         + jnp.dot(agg, wm_r[...], preferred_element_type=jnp.float32)
         + jnp.dot(c_r[...], wc_r[...], preferred_element_type=jnp.float32)
         + b_r[...])
    h = jnp.maximum(h, 0.0)
    ch = jnp.maximum(
        jnp.dot(c_r[...], wcc_r[...], preferred_element_type=jnp.float32)
        + bc_r[...], 0.0)
    h_r[...] = h
    ch_r[...] = ch
    g_r[...] = jnp.dot(h, wm2_r[...], preferred_element_type=jnp.float32)

  return pl.pallas_call(
      body,
      grid=(n // bn,),
      in_specs=[
          pl.BlockSpec((bn, 8), lambda i: (i, 0)),
          pl.BlockSpec((bn, 8), lambda i: (i, 0)),
          pl.BlockSpec((1, bn, 8), lambda i: (0, i, 0)),
          pl.BlockSpec((1, bn, 8), lambda i: (1, i, 0)),
          pl.BlockSpec((8, 16), lambda i: (0, 0)),
          pl.BlockSpec((8, 16), lambda i: (0, 0)),
          pl.BlockSpec((8, 16), lambda i: (0, 0)),
          pl.BlockSpec((1, 16), lambda i: (0, 0)),
          pl.BlockSpec((8, 16), lambda i: (0, 0)),
          pl.BlockSpec((1, 16), lambda i: (0, 0)),
          pl.BlockSpec((16, 8), lambda i: (0, 0)),
      ],
      out_specs=[
          pl.BlockSpec((bn, 16), lambda i: (i, 0)),
          pl.BlockSpec((bn, 16), lambda i: (i, 0)),
          pl.BlockSpec((bn, 8), lambda i: (i, 0)),
      ],
      out_shape=[
          jax.ShapeDtypeStruct((n, 16), jnp.float32),
          jax.ShapeDtypeStruct((n, 16), jnp.float32),
          jax.ShapeDtypeStruct((n, 8), jnp.float32),
      ],
  )


def _tc_layer2(n: int, bn: int):
  def body(h_r, ch_r, p0_r, p1_r, wx_r, wc_r, b_r, o_r):
    o_r[...] = (
        jnp.dot(h_r[...], wx_r[...], preferred_element_type=jnp.float32)
        + p0_r[0] + p1_r[0]
        + jnp.dot(ch_r[...], wc_r[...], preferred_element_type=jnp.float32)
        + b_r[...])

  return pl.pallas_call(
      body,
      grid=(n // bn,),
      in_specs=[
          pl.BlockSpec((bn, 16), lambda i: (i, 0)),
          pl.BlockSpec((bn, 16), lambda i: (i, 0)),
          pl.BlockSpec((1, bn, 8), lambda i: (0, i, 0)),
          pl.BlockSpec((1, bn, 8), lambda i: (1, i, 0)),
          pl.BlockSpec((16, 8), lambda i: (0, 0)),
          pl.BlockSpec((16, 8), lambda i: (0, 0)),
          pl.BlockSpec((1, 8), lambda i: (0, 0)),
      ],
      out_specs=[pl.BlockSpec((bn, 8), lambda i: (i, 0))],
      out_shape=[jax.ShapeDtypeStruct((n, 8), jnp.float32)],
  )


def kernel(x, c, edge_index, Wx1, Wm1, Wc1, b1, Wcc1, bc1,
           Wx2, Wm2, Wc2, b2, Wcc2, bc2):
  n = x.shape[0]
  e = edge_index.shape[1]
  npad = -(-(n + 16) // 128) * 128

  group = 2 * _B2
  per_worker_blocks = -(-(-(-e // _BLK) // _NW) // group) * group
  n_pairs = per_worker_blocks // group
  epad = per_worker_blocks * _NW * _BLK
  pad = epad - e
  ar = jnp.arange(pad, dtype=jnp.int32)
  src = jnp.concatenate([edge_index[0], ar % 997])
  dst = jnp.concatenate([edge_index[1], n + (ar % 8)])
  src2 = src.reshape(-1, _BLK)
  dst2 = dst.reshape(-1, _BLK)

  z8 = jnp.zeros((npad, 8), jnp.float32)
  sc8 = _make_edge_scatter(npad, 8, n_pairs)

  bn = 2000
  tc1 = _tc_layer1(n, bn)
  tc2 = _tc_layer2(n, bn)

  parts1 = sc8(x, src2, dst2, z8)
  h, ch, g = tc1(x, c, parts1, parts1, Wx1, Wm1, Wc1, b1.reshape(1, -1),
                 Wcc1, bc1.reshape(1, -1), Wm2)
  parts2 = sc8(g, src2, dst2, z8)
  (out,) = tc2(h, ch, parts2, parts2, Wx2, Wc2, b2.reshape(1, -1))
  return out


# speedup vs baseline: 45.5266x; 1.2065x over previous
import functools

import jax
import jax.numpy as jnp
from jax import lax
from jax.experimental import pallas as pl
from jax.experimental.pallas import tpu as pltpu
from jax.experimental.pallas import tpu_sc as plsc

_NCORES = 2
_NSUB = 16
_NW = _NCORES * _NSUB
_BLK = 128
_B2 = 8


def _make_edge_scatter(npad: int, feat: int, n_pairs: int):
  mesh = plsc.VectorSubcoreMesh(core_axis_name="c", subcore_axis_name="s")
  stripe = npad // _NSUB

  @functools.partial(
      pl.kernel,
      out_type=jax.ShapeDtypeStruct((_NCORES, npad, feat), jnp.float32),
      mesh=mesh,
      compiler_params=pltpu.CompilerParams(use_tc_tiling_on_sc=False),
      scratch_types=[
          pltpu.VMEM((4, _B2, 2, _BLK), jnp.int32),
          pltpu.VMEM((2, _B2, _BLK, feat), jnp.float32),
          pltpu.VMEM_SHARED((npad, feat), jnp.float32),
          pltpu.SemaphoreType.DMA,
          pltpu.SemaphoreType.DMA,
          pltpu.SemaphoreType.DMA,
      ],
  )
  def scatter_kernel(x_hbm, sd_hbm, z_hbm, out_hbm,
                     sd_v, rows_v, acc, isem, gsem, ssem):
    cid = lax.axis_index("c")
    sid = lax.axis_index("s")
    wid = cid * _NSUB + sid

    pltpu.sync_copy(z_hbm.at[pl.ds(sid * stripe, stripe)],
                    acc.at[pl.ds(sid * stripe, stripe)])
    plsc.subcore_barrier()

    base = wid * (n_pairs * 2 * _B2)

    def load_idx(islot, row):
      pltpu.async_copy(sd_hbm.at[pl.ds(row, _B2)], sd_v.at[islot], isem)

    def wait_idx(islot, row):
      pltpu.make_async_copy(sd_hbm.at[pl.ds(row, _B2)], sd_v.at[islot],
                            isem).wait()

    def fire_gathers(slot, islot):
      for j in range(_B2):
        pltpu.async_copy(x_hbm.at[sd_v.at[islot, j, 0]], rows_v.at[slot, j],
                         gsem)

    def wait_gathers(slot, islot):
      for j in range(_B2):
        pltpu.make_async_copy(x_hbm.at[sd_v.at[islot, j, 0]],
                              rows_v.at[slot, j], gsem).wait()

    def fire_scatters(slot, islot):
      for j in range(_B2):
        pltpu.async_copy(rows_v.at[slot, j], acc.at[sd_v.at[islot, j, 1]],
                         ssem, add=True)

    def drain_scatters(slot, islot):
      for j in range(_B2):
        pltpu.make_async_copy(rows_v.at[slot, j],
                              acc.at[sd_v.at[islot, j, 1]], ssem).wait()

    load_idx(0, base)
    load_idx(1, base + _B2)
    wait_idx(0, base)
    fire_gathers(0, 0)

    def body(p, carry):
      row = base + p * 2 * _B2
      i0 = (2 * p) % 4
      i1 = (2 * p + 1) % 4
      wait_gathers(0, i0)
      fire_scatters(0, i0)

      @pl.when(p > 0)
      def _():
        drain_scatters(1, (2 * p - 1) % 4)
      wait_idx(i1, row + _B2)
      fire_gathers(1, i1)

      @pl.when(p < n_pairs - 1)
      def _():
        load_idx((2 * p + 2) % 4, row + 2 * _B2)

      wait_gathers(1, i1)
      fire_scatters(1, i1)
      drain_scatters(0, i0)

      @pl.when(p < n_pairs - 1)
      def _():
        wait_idx((2 * p + 2) % 4, row + 2 * _B2)
        fire_gathers(0, (2 * p + 2) % 4)
        load_idx((2 * p + 3) % 4, row + 3 * _B2)
      return carry

    lax.fori_loop(0, n_pairs, body, 0)
    drain_scatters(1, (2 * n_pairs - 1) % 4)

    plsc.subcore_barrier()
    pltpu.sync_copy(acc.at[pl.ds(sid * stripe, stripe)],
                    out_hbm.at[cid, pl.ds(sid * stripe, stripe)])

  return scatter_kernel


def _tc_layer1(n: int, bn: int):
  def body(x_r, c_r, p0_r, p1_r, wx_r, wm_r, wc_r, b_r, wcc_r, bc_r, wm2_r,
           h_r, ch_r, g_r):
    agg = p0_r[0] + p1_r[0]
    h = (jnp.dot(x_r[...], wx_r[...], preferred_element_type=jnp.float32)
         + jnp.dot(agg, wm_r[...], preferred_element_type=jnp.float32)
         + jnp.dot(c_r[...], wc_r[...], preferred_element_type=jnp.float32)
         + b_r[...])
    h = jnp.maximum(h, 0.0)
    ch = jnp.maximum(
        jnp.dot(c_r[...], wcc_r[...], preferred_element_type=jnp.float32)
        + bc_r[...], 0.0)
    h_r[...] = h
    ch_r[...] = ch
    g_r[...] = jnp.dot(h, wm2_r[...], preferred_element_type=jnp.float32)

  return pl.pallas_call(
      body,
      grid=(n // bn,),
      in_specs=[
          pl.BlockSpec((bn, 8), lambda i: (i, 0)),
          pl.BlockSpec((bn, 8), lambda i: (i, 0)),
          pl.BlockSpec((1, bn, 8), lambda i: (0, i, 0)),
          pl.BlockSpec((1, bn, 8), lambda i: (1, i, 0)),
          pl.BlockSpec((8, 16), lambda i: (0, 0)),
          pl.BlockSpec((8, 16), lambda i: (0, 0)),
          pl.BlockSpec((8, 16), lambda i: (0, 0)),
          pl.BlockSpec((1, 16), lambda i: (0, 0)),
          pl.BlockSpec((8, 16), lambda i: (0, 0)),
          pl.BlockSpec((1, 16), lambda i: (0, 0)),
          pl.BlockSpec((16, 8), lambda i: (0, 0)),
      ],
      out_specs=[
          pl.BlockSpec((bn, 16), lambda i: (i, 0)),
          pl.BlockSpec((bn, 16), lambda i: (i, 0)),
          pl.BlockSpec((bn, 8), lambda i: (i, 0)),
      ],
      out_shape=[
          jax.ShapeDtypeStruct((n, 16), jnp.float32),
          jax.ShapeDtypeStruct((n, 16), jnp.float32),
          jax.ShapeDtypeStruct((n, 8), jnp.float32),
      ],
  )


def _tc_layer2(n: int, bn: int):
  def body(h_r, ch_r, p0_r, p1_r, wx_r, wc_r, b_r, o_r):
    o_r[...] = (
        jnp.dot(h_r[...], wx_r[...], preferred_element_type=jnp.float32)
        + p0_r[0] + p1_r[0]
        + jnp.dot(ch_r[...], wc_r[...], preferred_element_type=jnp.float32)
        + b_r[...])

  return pl.pallas_call(
      body,
      grid=(n // bn,),
      in_specs=[
          pl.BlockSpec((bn, 16), lambda i: (i, 0)),
          pl.BlockSpec((bn, 16), lambda i: (i, 0)),
          pl.BlockSpec((1, bn, 8), lambda i: (0, i, 0)),
          pl.BlockSpec((1, bn, 8), lambda i: (1, i, 0)),
          pl.BlockSpec((16, 8), lambda i: (0, 0)),
          pl.BlockSpec((16, 8), lambda i: (0, 0)),
          pl.BlockSpec((1, 8), lambda i: (0, 0)),
      ],
      out_specs=[pl.BlockSpec((bn, 8), lambda i: (i, 0))],
      out_shape=[jax.ShapeDtypeStruct((n, 8), jnp.float32)],
  )


def kernel(x, c, edge_index, Wx1, Wm1, Wc1, b1, Wcc1, bc1,
           Wx2, Wm2, Wc2, b2, Wcc2, bc2):
  n = x.shape[0]
  e = edge_index.shape[1]
  npad = -(-(n + 16) // 128) * 128

  group = 2 * _B2
  per_worker_blocks = -(-(-(-e // _BLK) // _NW) // group) * group
  n_pairs = per_worker_blocks // group
  epad = per_worker_blocks * _NW * _BLK
  pad = epad - e
  ar = jnp.arange(pad, dtype=jnp.int32)
  src = jnp.concatenate([edge_index[0], ar % 997])
  dst = jnp.concatenate([edge_index[1], n + (ar % 8)])
  sd = jnp.stack([src.reshape(-1, _BLK), dst.reshape(-1, _BLK)], axis=1)

  z8 = jnp.zeros((npad, 8), jnp.float32)
  sc8 = _make_edge_scatter(npad, 8, n_pairs)

  bn = 2000
  tc1 = _tc_layer1(n, bn)
  tc2 = _tc_layer2(n, bn)

  parts1 = sc8(x, sd, z8)
  h, ch, g = tc1(x, c, parts1, parts1, Wx1, Wm1, Wc1, b1.reshape(1, -1),
                 Wcc1, bc1.reshape(1, -1), Wm2)
  parts2 = sc8(g, sd, z8)
  (out,) = tc2(h, ch, parts2, parts2, Wx2, Wc2, b2.reshape(1, -1))
  return out


# speedup vs baseline: 48.8076x; 1.0721x over previous
import functools

import jax
import jax.numpy as jnp
from jax import lax
from jax.experimental import pallas as pl
from jax.experimental.pallas import tpu as pltpu
from jax.experimental.pallas import tpu_sc as plsc

_NCORES = 2
_NSUB = 16
_NW = _NCORES * _NSUB
_BLK = 128
_B2 = 16


def _make_edge_scatter(npad: int, feat: int, n_pairs: int):
  mesh = plsc.VectorSubcoreMesh(core_axis_name="c", subcore_axis_name="s")
  stripe = npad // _NSUB

  @functools.partial(
      pl.kernel,
      out_type=jax.ShapeDtypeStruct((_NCORES, npad, feat), jnp.float32),
      mesh=mesh,
      compiler_params=pltpu.CompilerParams(use_tc_tiling_on_sc=False),
      scratch_types=[
          pltpu.VMEM((4, _B2, 2, _BLK), jnp.int32),
          pltpu.VMEM((2, _B2, _BLK, feat), jnp.float32),
          pltpu.VMEM_SHARED((npad, feat), jnp.float32),
          pltpu.SemaphoreType.DMA,
          pltpu.SemaphoreType.DMA,
          pltpu.SemaphoreType.DMA,
      ],
  )
  def scatter_kernel(x_hbm, sd_hbm, z_hbm, out_hbm,
                     sd_v, rows_v, acc, isem, gsem, ssem):
    cid = lax.axis_index("c")
    sid = lax.axis_index("s")
    wid = cid * _NSUB + sid

    pltpu.sync_copy(z_hbm.at[pl.ds(sid * stripe, stripe)],
                    acc.at[pl.ds(sid * stripe, stripe)])
    plsc.subcore_barrier()

    base = wid * (n_pairs * 2 * _B2)

    def load_idx(islot, row):
      pltpu.async_copy(sd_hbm.at[pl.ds(row, _B2)], sd_v.at[islot], isem)

    def wait_idx(islot, row):
      pltpu.make_async_copy(sd_hbm.at[pl.ds(row, _B2)], sd_v.at[islot],
                            isem).wait()

    def fire_gathers(slot, islot):
      for j in range(_B2):
        pltpu.async_copy(x_hbm.at[sd_v.at[islot, j, 0]], rows_v.at[slot, j],
                         gsem)

    def wait_gathers(slot, islot):
      for j in range(_B2):
        pltpu.make_async_copy(x_hbm.at[sd_v.at[islot, j, 0]],
                              rows_v.at[slot, j], gsem).wait()

    def fire_scatters(slot, islot):
      for j in range(_B2):
        pltpu.async_copy(rows_v.at[slot, j], acc.at[sd_v.at[islot, j, 1]],
                         ssem, add=True)

    def drain_scatters(slot, islot):
      for j in range(_B2):
        pltpu.make_async_copy(rows_v.at[slot, j],
                              acc.at[sd_v.at[islot, j, 1]], ssem).wait()

    load_idx(0, base)
    load_idx(1, base + _B2)
    wait_idx(0, base)
    fire_gathers(0, 0)

    def body(p, carry):
      row = base + p * 2 * _B2
      i0 = (2 * p) % 4
      i1 = (2 * p + 1) % 4
      wait_gathers(0, i0)
      fire_scatters(0, i0)

      @pl.when(p > 0)
      def _():
        drain_scatters(1, (2 * p - 1) % 4)
      wait_idx(i1, row + _B2)
      fire_gathers(1, i1)

      @pl.when(p < n_pairs - 1)
      def _():
        load_idx((2 * p + 2) % 4, row + 2 * _B2)

      wait_gathers(1, i1)
      fire_scatters(1, i1)
      drain_scatters(0, i0)

      @pl.when(p < n_pairs - 1)
      def _():
        wait_idx((2 * p + 2) % 4, row + 2 * _B2)
        fire_gathers(0, (2 * p + 2) % 4)
        load_idx((2 * p + 3) % 4, row + 3 * _B2)
      return carry

    lax.fori_loop(0, n_pairs, body, 0)
    drain_scatters(1, (2 * n_pairs - 1) % 4)

    plsc.subcore_barrier()
    pltpu.sync_copy(acc.at[pl.ds(sid * stripe, stripe)],
                    out_hbm.at[cid, pl.ds(sid * stripe, stripe)])

  return scatter_kernel


def _tc_layer1(n: int, bn: int):
  def body(x_r, c_r, p0_r, p1_r, wx_r, wm_r, wc_r, b_r, wcc_r, bc_r, wm2_r,
           h_r, ch_r, g_r):
    agg = p0_r[0] + p1_r[0]
    h = (jnp.dot(x_r[...], wx_r[...], preferred_element_type=jnp.float32)
         + jnp.dot(agg, wm_r[...], preferred_element_type=jnp.float32)
         + jnp.dot(c_r[...], wc_r[...], preferred_element_type=jnp.float32)
         + b_r[...])
    h = jnp.maximum(h, 0.0)
    ch = jnp.maximum(
        jnp.dot(c_r[...], wcc_r[...], preferred_element_type=jnp.float32)
        + bc_r[...], 0.0)
    h_r[...] = h
    ch_r[...] = ch
    g_r[...] = jnp.dot(h, wm2_r[...], preferred_element_type=jnp.float32)

  return pl.pallas_call(
      body,
      grid=(n // bn,),
      in_specs=[
          pl.BlockSpec((bn, 8), lambda i: (i, 0)),
          pl.BlockSpec((bn, 8), lambda i: (i, 0)),
          pl.BlockSpec((1, bn, 8), lambda i: (0, i, 0)),
          pl.BlockSpec((1, bn, 8), lambda i: (1, i, 0)),
          pl.BlockSpec((8, 16), lambda i: (0, 0)),
          pl.BlockSpec((8, 16), lambda i: (0, 0)),
          pl.BlockSpec((8, 16), lambda i: (0, 0)),
          pl.BlockSpec((1, 16), lambda i: (0, 0)),
          pl.BlockSpec((8, 16), lambda i: (0, 0)),
          pl.BlockSpec((1, 16), lambda i: (0, 0)),
          pl.BlockSpec((16, 8), lambda i: (0, 0)),
      ],
      out_specs=[
          pl.BlockSpec((bn, 16), lambda i: (i, 0)),
          pl.BlockSpec((bn, 16), lambda i: (i, 0)),
          pl.BlockSpec((bn, 8), lambda i: (i, 0)),
      ],
      out_shape=[
          jax.ShapeDtypeStruct((n, 16), jnp.float32),
          jax.ShapeDtypeStruct((n, 16), jnp.float32),
          jax.ShapeDtypeStruct((n, 8), jnp.float32),
      ],
  )


def _tc_layer2(n: int, bn: int):
  def body(h_r, ch_r, p0_r, p1_r, wx_r, wc_r, b_r, o_r):
    o_r[...] = (
        jnp.dot(h_r[...], wx_r[...], preferred_element_type=jnp.float32)
        + p0_r[0] + p1_r[0]
        + jnp.dot(ch_r[...], wc_r[...], preferred_element_type=jnp.float32)
        + b_r[...])

  return pl.pallas_call(
      body,
      grid=(n // bn,),
      in_specs=[
          pl.BlockSpec((bn, 16), lambda i: (i, 0)),
          pl.BlockSpec((bn, 16), lambda i: (i, 0)),
          pl.BlockSpec((1, bn, 8), lambda i: (0, i, 0)),
          pl.BlockSpec((1, bn, 8), lambda i: (1, i, 0)),
          pl.BlockSpec((16, 8), lambda i: (0, 0)),
          pl.BlockSpec((16, 8), lambda i: (0, 0)),
          pl.BlockSpec((1, 8), lambda i: (0, 0)),
      ],
      out_specs=[pl.BlockSpec((bn, 8), lambda i: (i, 0))],
      out_shape=[jax.ShapeDtypeStruct((n, 8), jnp.float32)],
  )


def kernel(x, c, edge_index, Wx1, Wm1, Wc1, b1, Wcc1, bc1,
           Wx2, Wm2, Wc2, b2, Wcc2, bc2):
  n = x.shape[0]
  e = edge_index.shape[1]
  npad = -(-(n + 16) // 128) * 128

  group = 2 * _B2
  per_worker_blocks = -(-(-(-e // _BLK) // _NW) // group) * group
  n_pairs = per_worker_blocks // group
  epad = per_worker_blocks * _NW * _BLK
  pad = epad - e
  ar = jnp.arange(pad, dtype=jnp.int32)
  src = jnp.concatenate([edge_index[0], ar % 997])
  dst = jnp.concatenate([edge_index[1], n + (ar % 8)])
  sd = jnp.stack([src.reshape(-1, _BLK), dst.reshape(-1, _BLK)], axis=1)

  z8 = jnp.zeros((npad, 8), jnp.float32)
  sc8 = _make_edge_scatter(npad, 8, n_pairs)

  bn = 2000
  tc1 = _tc_layer1(n, bn)
  tc2 = _tc_layer2(n, bn)

  parts1 = sc8(x, sd, z8)
  h, ch, g = tc1(x, c, parts1, parts1, Wx1, Wm1, Wc1, b1.reshape(1, -1),
                 Wcc1, bc1.reshape(1, -1), Wm2)
  parts2 = sc8(g, sd, z8)
  (out,) = tc2(h, ch, parts2, parts2, Wx2, Wc2, b2.reshape(1, -1))
  return out


# speedup vs baseline: 48.8184x; 1.0002x over previous
import functools

import jax
import jax.numpy as jnp
from jax import lax
from jax.experimental import pallas as pl
from jax.experimental.pallas import tpu as pltpu
from jax.experimental.pallas import tpu_sc as plsc

_NCORES = 2
_NSUB = 16
_NW = _NCORES * _NSUB
_BLK = 128
_B2 = 24


def _make_edge_scatter(npad: int, feat: int, n_pairs: int):
  mesh = plsc.VectorSubcoreMesh(core_axis_name="c", subcore_axis_name="s")
  stripe = npad // _NSUB

  @functools.partial(
      pl.kernel,
      out_type=jax.ShapeDtypeStruct((_NCORES, npad, feat), jnp.float32),
      mesh=mesh,
      compiler_params=pltpu.CompilerParams(use_tc_tiling_on_sc=False),
      scratch_types=[
          pltpu.VMEM((4, _B2, 2, _BLK), jnp.int32),
          pltpu.VMEM((2, _B2, _BLK, feat), jnp.float32),
          pltpu.VMEM_SHARED((npad, feat), jnp.float32),
          pltpu.SemaphoreType.DMA,
          pltpu.SemaphoreType.DMA,
          pltpu.SemaphoreType.DMA,
      ],
  )
  def scatter_kernel(x_hbm, sd_hbm, z_hbm, out_hbm,
                     sd_v, rows_v, acc, isem, gsem, ssem):
    cid = lax.axis_index("c")
    sid = lax.axis_index("s")
    wid = cid * _NSUB + sid

    pltpu.sync_copy(z_hbm.at[pl.ds(sid * stripe, stripe)],
                    acc.at[pl.ds(sid * stripe, stripe)])
    plsc.subcore_barrier()

    base = wid * (n_pairs * 2 * _B2)

    def load_idx(islot, row):
      pltpu.async_copy(sd_hbm.at[pl.ds(row, _B2)], sd_v.at[islot], isem)

    def wait_idx(islot, row):
      pltpu.make_async_copy(sd_hbm.at[pl.ds(row, _B2)], sd_v.at[islot],
                            isem).wait()

    def fire_gathers(slot, islot):
      for j in range(_B2):
        pltpu.async_copy(x_hbm.at[sd_v.at[islot, j, 0]], rows_v.at[slot, j],
                         gsem)

    def wait_gathers(slot, islot):
      for j in range(_B2):
        pltpu.make_async_copy(x_hbm.at[sd_v.at[islot, j, 0]],
                              rows_v.at[slot, j], gsem).wait()

    def fire_scatters(slot, islot):
      for j in range(_B2):
        pltpu.async_copy(rows_v.at[slot, j], acc.at[sd_v.at[islot, j, 1]],
                         ssem, add=True)

    def drain_scatters(slot, islot):
      for j in range(_B2):
        pltpu.make_async_copy(rows_v.at[slot, j],
                              acc.at[sd_v.at[islot, j, 1]], ssem).wait()

    load_idx(0, base)
    load_idx(1, base + _B2)
    wait_idx(0, base)
    fire_gathers(0, 0)

    def body(p, carry):
      row = base + p * 2 * _B2
      i0 = (2 * p) % 4
      i1 = (2 * p + 1) % 4
      wait_gathers(0, i0)
      fire_scatters(0, i0)

      @pl.when(p > 0)
      def _():
        drain_scatters(1, (2 * p - 1) % 4)
      wait_idx(i1, row + _B2)
      fire_gathers(1, i1)

      @pl.when(p < n_pairs - 1)
      def _():
        load_idx((2 * p + 2) % 4, row + 2 * _B2)

      wait_gathers(1, i1)
      fire_scatters(1, i1)
      drain_scatters(0, i0)

      @pl.when(p < n_pairs - 1)
      def _():
        wait_idx((2 * p + 2) % 4, row + 2 * _B2)
        fire_gathers(0, (2 * p + 2) % 4)
        load_idx((2 * p + 3) % 4, row + 3 * _B2)
      return carry

    lax.fori_loop(0, n_pairs, body, 0)
    drain_scatters(1, (2 * n_pairs - 1) % 4)

    plsc.subcore_barrier()
    pltpu.sync_copy(acc.at[pl.ds(sid * stripe, stripe)],
                    out_hbm.at[cid, pl.ds(sid * stripe, stripe)])

  return scatter_kernel


def _tc_layer1(n: int, bn: int):
  def body(x_r, c_r, p0_r, p1_r, wx_r, wm_r, wc_r, b_r, wcc_r, bc_r, wm2_r,
           h_r, ch_r, g_r):
    agg = p0_r[0] + p1_r[0]
    h = (jnp.dot(x_r[...], wx_r[...], preferred_element_type=jnp.float32)
         + jnp.dot(agg, wm_r[...], preferred_element_type=jnp.float32)
         + jnp.dot(c_r[...], wc_r[...], preferred_element_type=jnp.float32)
         + b_r[...])
    h = jnp.maximum(h, 0.0)
    ch = jnp.maximum(
        jnp.dot(c_r[...], wcc_r[...], preferred_element_type=jnp.float32)
        + bc_r[...], 0.0)
    h_r[...] = h
    ch_r[...] = ch
    g_r[...] = jnp.dot(h, wm2_r[...], preferred_element_type=jnp.float32)

  return pl.pallas_call(
      body,
      grid=(n // bn,),
      in_specs=[
          pl.BlockSpec((bn, 8), lambda i: (i, 0)),
          pl.BlockSpec((bn, 8), lambda i: (i, 0)),
          pl.BlockSpec((1, bn, 8), lambda i: (0, i, 0)),
          pl.BlockSpec((1, bn, 8), lambda i: (1, i, 0)),
          pl.BlockSpec((8, 16), lambda i: (0, 0)),
          pl.BlockSpec((8, 16), lambda i: (0, 0)),
          pl.BlockSpec((8, 16), lambda i: (0, 0)),
          pl.BlockSpec((1, 16), lambda i: (0, 0)),
          pl.BlockSpec((8, 16), lambda i: (0, 0)),
          pl.BlockSpec((1, 16), lambda i: (0, 0)),
          pl.BlockSpec((16, 8), lambda i: (0, 0)),
      ],
      out_specs=[
          pl.BlockSpec((bn, 16), lambda i: (i, 0)),
          pl.BlockSpec((bn, 16), lambda i: (i, 0)),
          pl.BlockSpec((bn, 8), lambda i: (i, 0)),
      ],
      out_shape=[
          jax.ShapeDtypeStruct((n, 16), jnp.float32),
          jax.ShapeDtypeStruct((n, 16), jnp.float32),
          jax.ShapeDtypeStruct((n, 8), jnp.float32),
      ],
  )


def _tc_layer2(n: int, bn: int):
  def body(h_r, ch_r, p0_r, p1_r, wx_r, wc_r, b_r, o_r):
    o_r[...] = (
        jnp.dot(h_r[...], wx_r[...], preferred_element_type=jnp.float32)
        + p0_r[0] + p1_r[0]
        + jnp.dot(ch_r[...], wc_r[...], preferred_element_type=jnp.float32)
        + b_r[...])

  return pl.pallas_call(
      body,
      grid=(n // bn,),
      in_specs=[
          pl.BlockSpec((bn, 16), lambda i: (i, 0)),
          pl.BlockSpec((bn, 16), lambda i: (i, 0)),
          pl.BlockSpec((1, bn, 8), lambda i: (0, i, 0)),
          pl.BlockSpec((1, bn, 8), lambda i: (1, i, 0)),
          pl.BlockSpec((16, 8), lambda i: (0, 0)),
          pl.BlockSpec((16, 8), lambda i: (0, 0)),
          pl.BlockSpec((1, 8), lambda i: (0, 0)),
      ],
      out_specs=[pl.BlockSpec((bn, 8), lambda i: (i, 0))],
      out_shape=[jax.ShapeDtypeStruct((n, 8), jnp.float32)],
  )


def kernel(x, c, edge_index, Wx1, Wm1, Wc1, b1, Wcc1, bc1,
           Wx2, Wm2, Wc2, b2, Wcc2, bc2):
  n = x.shape[0]
  e = edge_index.shape[1]
  npad = -(-(n + 16) // 128) * 128

  group = 2 * _B2
  per_worker_blocks = -(-(-(-e // _BLK) // _NW) // group) * group
  n_pairs = per_worker_blocks // group
  epad = per_worker_blocks * _NW * _BLK
  pad = epad - e
  ar = jnp.arange(pad, dtype=jnp.int32)
  src = jnp.concatenate([edge_index[0], ar % 997])
  dst = jnp.concatenate([edge_index[1], n + (ar % 8)])
  sd = jnp.stack([src.reshape(-1, _BLK), dst.reshape(-1, _BLK)], axis=1)

  z8 = jnp.zeros((npad, 8), jnp.float32)
  sc8 = _make_edge_scatter(npad, 8, n_pairs)

  bn = 2000
  tc1 = _tc_layer1(n, bn)
  tc2 = _tc_layer2(n, bn)

  parts1 = sc8(x, sd, z8)
  h, ch, g = tc1(x, c, parts1, parts1, Wx1, Wm1, Wc1, b1.reshape(1, -1),
                 Wcc1, bc1.reshape(1, -1), Wm2)
  parts2 = sc8(g, sd, z8)
  (out,) = tc2(h, ch, parts2, parts2, Wx2, Wc2, b2.reshape(1, -1))
  return out
